# baseline (speedup 1.0000x reference)
import functools

import jax
import jax.numpy as jnp
from jax import lax
from jax.experimental import pallas as pl
from jax.experimental.pallas import tpu as pltpu
from jax.experimental.pallas import tpu_sc as plsc

VOCAB = 100000
D = 768
MAXLEN = 2048
B = 4
S = 2048
L = 16
NC, NS = 2, 16
NW = NC * NS
SPW = S // NW
DJ = D // L
TG = 4

_mesh = plsc.VectorSubcoreMesh(core_axis_name="c", subcore_axis_name="s")


def _rsqrt_newton(x):
    i = plsc.bitcast(x, jnp.int32)
    i = jnp.int32(0x5F3759DF) - lax.shift_right_logical(i, 1)
    y = plsc.bitcast(i, jnp.float32)
    half = x * 0.5
    for _ in range(3):
        y = y * (1.5 - half * y * y)
    return y


@functools.partial(
    pl.kernel,
    mesh=_mesh,
    out_type=jax.ShapeDtypeStruct((B * S, D), jnp.float32),
    compiler_params=pltpu.CompilerParams(needs_layout_passes=False),
    scratch_types=[
        pltpu.VMEM((SPW, D), jnp.float32),
        pltpu.VMEM((SPW, D), jnp.float32),
        pltpu.VMEM((D,), jnp.float32),
        pltpu.VMEM((D,), jnp.float32),
        pltpu.VMEM((2, D), jnp.float32),
        pltpu.VMEM((SPW,), jnp.int32),
        pltpu.VMEM((SPW + L,), jnp.float32),
        pltpu.SemaphoreType.DMA,
    ],
)
def _emb_kernel(x_hbm, segf_hbm, tok_hbm, pos_hbm, segtab_hbm, gamma_hbm,
                beta_hbm, out_hbm, pos_v, tok_v, g_v, b_v, st_v,
                idx_v, segf_v, sem):
    wid = lax.axis_index("s") * NC + lax.axis_index("c")
    s0 = wid * SPW

    pltpu.sync_copy(pos_hbm.at[pl.ds(s0, SPW)], pos_v)
    pltpu.sync_copy(segtab_hbm, st_v)
    pltpu.sync_copy(gamma_hbm, g_v)
    pltpu.sync_copy(beta_hbm, b_v)

    def batch_body(b, _):
        base = pl.multiple_of(b * S + s0, SPW)
        pltpu.sync_copy(x_hbm.at[pl.ds(base, SPW)], idx_v)
        pltpu.sync_copy(segf_hbm.at[pl.ds(base, SPW)],
                        segf_v.at[pl.ds(0, SPW)])
        pltpu.async_copy(tok_hbm.at[idx_v], tok_v, sem).wait()

        def group_body(tg, _):
            t0 = tg * TG
            fs = []
            for i in range(TG):
                fv = segf_v[pl.ds(t0 + i, L)]
                fs.append(jnp.full((L,), fv[0], dtype=jnp.float32))
            zeros = jnp.zeros((L,), jnp.float32)
            sums = [zeros] * TG
            sqs = [zeros] * TG

            for j in range(DJ):
                sl = pl.ds(j * L, L)
                sg0 = st_v[0, sl]
                sgd = st_v[1, sl] - sg0
                for i in range(TG):
                    v = (tok_v[t0 + i, sl] + pos_v[t0 + i, sl]) + \
                        (sg0 + fs[i] * sgd)
                    tok_v[t0 + i, sl] = v
                    sums[i] = sums[i] + v
                    sqs[i] = sqs[i] + v * v

            means = []
            rs = []
            for i in range(TG):
                s1 = jnp.sum(sums[i])
                s2 = jnp.sum(sqs[i])
                mean = s1 * (1.0 / D)
                var = s2 * (1.0 / D) - mean * mean
                means.append(jnp.full((L,), mean, dtype=jnp.float32))
                rs.append(_rsqrt_newton(
                    jnp.full((L,), var + 1e-5, dtype=jnp.float32)))

            for j in range(DJ):
                sl = pl.ds(j * L, L)
                g = g_v[sl]
                bb = b_v[sl]
                for i in range(TG):
                    h = tok_v[t0 + i, sl]
                    tok_v[t0 + i, sl] = (h - means[i]) * rs[i] * g + bb

            return 0
        lax.fori_loop(0, SPW // TG, group_body, 0)

        pltpu.sync_copy(tok_v, out_hbm.at[pl.ds(base, SPW)])
        return 0
    lax.fori_loop(0, B, batch_body, 0)


def kernel(x, seg, tok_embed, pos_embed, seg_embed, gamma, beta):
    x_flat = x.reshape(-1).astype(jnp.int32)
    segf = seg.reshape(-1).astype(jnp.float32)
    out = _emb_kernel(x_flat, segf, tok_embed, pos_embed, seg_embed,
                      gamma, beta)
    return out.reshape(B, S, D)


# speedup vs baseline: 3.8167x; 3.8167x over previous
import functools

import jax
import jax.numpy as jnp
from jax import lax
from jax.experimental import pallas as pl
from jax.experimental.pallas import tpu as pltpu
from jax.experimental.pallas import tpu_sc as plsc

VOCAB = 100000
D = 768
MAXLEN = 2048
B = 4
S = 2048
L = 16
NC, NS = 2, 16
NW = NC * NS
SPW = S // NW
CH = 32
NCH = B * SPW // CH
DJ = D // L
TG = 4

_mesh = plsc.VectorSubcoreMesh(core_axis_name="c", subcore_axis_name="s")


def _rsqrt_newton(x):
    i = plsc.bitcast(x, jnp.int32)
    i = jnp.int32(0x5F3759DF) - lax.shift_right_logical(i, 1)
    y = plsc.bitcast(i, jnp.float32)
    half = x * 0.5
    for _ in range(3):
        y = y * (1.5 - half * y * y)
    return y


@functools.partial(
    pl.kernel,
    mesh=_mesh,
    out_type=jax.ShapeDtypeStruct((B * S, D), jnp.float32),
    compiler_params=pltpu.CompilerParams(needs_layout_passes=False),
    scratch_types=[
        pltpu.VMEM((SPW, D), jnp.float32),
        pltpu.VMEM((CH, D), jnp.float32),
        pltpu.VMEM((CH, D), jnp.float32),
        pltpu.VMEM((D,), jnp.float32),
        pltpu.VMEM((D,), jnp.float32),
        pltpu.VMEM((2, D), jnp.float32),
        pltpu.VMEM((CH,), jnp.int32),
        pltpu.VMEM((CH + L,), jnp.float32),
        pltpu.SemaphoreType.DMA,
    ],
)
def _emb_kernel(x_hbm, segf_hbm, tok_hbm, pos_hbm, segtab_hbm, gamma_hbm,
                beta_hbm, out_hbm, pos_v, tok_v, h_v, g_v, b_v, st_v,
                idx_v, segf_v, sem):
    wid = lax.axis_index("s") * NC + lax.axis_index("c")
    s0 = wid * SPW

    pltpu.sync_copy(pos_hbm.at[pl.ds(s0, SPW)], pos_v)
    pltpu.sync_copy(segtab_hbm, st_v)
    pltpu.sync_copy(gamma_hbm, g_v)
    pltpu.sync_copy(beta_hbm, b_v)

    def chunk_body(c, _):
        b = c // 2
        half = c % 2
        base = pl.multiple_of(b * S + s0 + half * CH, CH)
        prow = half * CH

        pltpu.sync_copy(x_hbm.at[pl.ds(base, CH)], idx_v)
        pltpu.sync_copy(segf_hbm.at[pl.ds(base, CH)],
                        segf_v.at[pl.ds(0, CH)])
        pltpu.async_copy(tok_hbm.at[idx_v], tok_v, sem).wait()

        def group_body(tg, _):
            t0 = tg * TG
            fs = []
            for i in range(TG):
                fv = segf_v[pl.ds(t0 + i, L)]
                fs.append(jnp.full((L,), fv[0], dtype=jnp.float32))
            zeros = jnp.zeros((L,), jnp.float32)

            @plsc.parallel_loop(0, DJ, unroll=4, carry=(zeros,) * (2 * TG))
            def acc(j, carry):
                sl = pl.ds(j * L, L)
                sg0 = st_v[0, sl]
                sgd = st_v[1, sl] - sg0
                nxt = []
                for i in range(TG):
                    v = (tok_v[t0 + i, sl] + pos_v[prow + t0 + i, sl]) + \
                        (sg0 + fs[i] * sgd)
                    h_v[t0 + i, sl] = v
                    nxt.append(carry[i] + v)
                    nxt.append(carry[TG + i] + v * v)
                return tuple(nxt[0::2]) + tuple(nxt[1::2])

            means = []
            rs = []
            for i in range(TG):
                s1 = jnp.sum(acc[i])
                s2 = jnp.sum(acc[TG + i])
                mean = s1 * (1.0 / D)
                var = s2 * (1.0 / D) - mean * mean
                means.append(jnp.full((L,), mean, dtype=jnp.float32))
                rs.append(_rsqrt_newton(
                    jnp.full((L,), var + 1e-5, dtype=jnp.float32)))

            @plsc.parallel_loop(0, DJ, unroll=4)
            def norm(j):
                sl = pl.ds(j * L, L)
                g = g_v[sl]
                bb = b_v[sl]
                for i in range(TG):
                    h = h_v[t0 + i, sl]
                    tok_v[t0 + i, sl] = (h - means[i]) * rs[i] * g + bb

            return 0
        lax.fori_loop(0, CH // TG, group_body, 0)

        pltpu.sync_copy(tok_v, out_hbm.at[pl.ds(base, CH)])
        return 0
    lax.fori_loop(0, NCH, chunk_body, 0)


def kernel(x, seg, tok_embed, pos_embed, seg_embed, gamma, beta):
    x_flat = x.reshape(-1).astype(jnp.int32)
    segf = seg.reshape(-1).astype(jnp.float32)
    out = _emb_kernel(x_flat, segf, tok_embed, pos_embed, seg_embed,
                      gamma, beta)
    return out.reshape(B, S, D)


# speedup vs baseline: 4.8271x; 1.2647x over previous
import functools

import jax
import jax.numpy as jnp
from jax import lax
from jax.experimental import pallas as pl
from jax.experimental.pallas import tpu as pltpu
from jax.experimental.pallas import tpu_sc as plsc

VOCAB = 100000
D = 768
B = 4
S = 2048
L = 16
NC, NS = 2, 16
NW = NC * NS
SPW = S // NW
CH = 32
DJ = D // L
TG = 4

_mesh = plsc.VectorSubcoreMesh(core_axis_name="c", subcore_axis_name="s")


def _rsqrt_newton(x):
    i = plsc.bitcast(x, jnp.int32)
    i = jnp.int32(0x5F3759DF) - lax.shift_right_logical(i, 1)
    y = plsc.bitcast(i, jnp.float32)
    half = x * 0.5
    for _ in range(3):
        y = y * (1.5 - half * y * y)
    return y


@functools.partial(
    pl.kernel,
    mesh=_mesh,
    out_type=jax.ShapeDtypeStruct((B * S, D), jnp.float32),
    compiler_params=pltpu.CompilerParams(needs_layout_passes=False),
    scratch_types=[
        pltpu.VMEM((SPW, D), jnp.float32),
        pltpu.VMEM((CH, D), jnp.float32),
        pltpu.VMEM((CH, D), jnp.float32),
        pltpu.VMEM((CH, D), jnp.float32),
        pltpu.VMEM((D,), jnp.float32),
        pltpu.VMEM((D,), jnp.float32),
        pltpu.VMEM((2, D), jnp.float32),
        pltpu.VMEM((CH,), jnp.int32),
        pltpu.VMEM((CH,), jnp.int32),
        pltpu.VMEM((CH + L,), jnp.float32),
        pltpu.VMEM((CH + L,), jnp.float32),
        pltpu.SemaphoreType.DMA,
        pltpu.SemaphoreType.DMA,
        pltpu.SemaphoreType.DMA,
        pltpu.SemaphoreType.DMA,
    ],
)
def _emb_kernel(x_hbm, segf_hbm, tok_hbm, pos_hbm, segtab_hbm, gamma_hbm,
                beta_hbm, out_hbm, pos_v, tok0, tok1, h_v, g_v, b_v, st_v,
                idx0, idx1, segf0, segf1, gsem0, gsem1, osem0, osem1):
    wid = lax.axis_index("s") * NC + lax.axis_index("c")
    s0 = wid * SPW

    pltpu.sync_copy(pos_hbm.at[pl.ds(s0, SPW)], pos_v)
    pltpu.sync_copy(segtab_hbm, st_v)
    pltpu.sync_copy(gamma_hbm, g_v)
    pltpu.sync_copy(beta_hbm, b_v)

    def compute_chunk(tok_v, segf_v, prow):
        def group_body(tg, _):
            t0 = tg * TG
            fs = []
            for i in range(TG):
                fv = segf_v[pl.ds(t0 + i, L)]
                fs.append(jnp.full((L,), fv[0], dtype=jnp.float32))
            zeros = jnp.zeros((L,), jnp.float32)

            @plsc.parallel_loop(0, DJ, unroll=4, carry=(zeros,) * (2 * TG))
            def acc(j, carry):
                sl = pl.ds(j * L, L)
                sg0 = st_v[0, sl]
                sgd = st_v[1, sl] - sg0
                nxt = []
                for i in range(TG):
                    v = (tok_v[t0 + i, sl] + pos_v[prow + t0 + i, sl]) + \
                        (sg0 + fs[i] * sgd)
                    h_v[t0 + i, sl] = v
                    nxt.append(carry[i] + v)
                    nxt.append(carry[TG + i] + v * v)
                return tuple(nxt[0::2]) + tuple(nxt[1::2])

            means = []
            rs = []
            for i in range(TG):
                s1 = jnp.sum(acc[i])
                s2 = jnp.sum(acc[TG + i])
                mean = s1 * (1.0 / D)
                var = s2 * (1.0 / D) - mean * mean
                means.append(jnp.full((L,), mean, dtype=jnp.float32))
                rs.append(_rsqrt_newton(
                    jnp.full((L,), var + 1e-5, dtype=jnp.float32)))

            @plsc.parallel_loop(0, DJ, unroll=4)
            def norm(j):
                sl = pl.ds(j * L, L)
                g = g_v[sl]
                bb = b_v[sl]
                for i in range(TG):
                    h = h_v[t0 + i, sl]
                    tok_v[t0 + i, sl] = (h - means[i]) * rs[i] * g + bb

            return 0
        lax.fori_loop(0, CH // TG, group_body, 0)

    def wait_gather(tok_v, gsem):
        pltpu.make_async_copy(tok_hbm.at[pl.ds(0, CH)], tok_v, gsem).wait()

    def wait_scatter(tok_v, osem):
        pltpu.make_async_copy(tok_v, out_hbm.at[pl.ds(0, CH)], osem).wait()

    base00 = pl.multiple_of(s0, CH)
    pltpu.sync_copy(x_hbm.at[pl.ds(base00, CH)], idx0)
    pltpu.sync_copy(segf_hbm.at[pl.ds(base00, CH)], segf0.at[pl.ds(0, CH)])
    pltpu.make_async_copy(tok_hbm.at[idx0], tok0, gsem0).start()

    def pair_body(p, _):
        baseA = pl.multiple_of(p * S + s0, CH)
        baseB = pl.multiple_of(baseA + CH, CH)

        pltpu.sync_copy(x_hbm.at[pl.ds(baseB, CH)], idx1)
        pltpu.sync_copy(segf_hbm.at[pl.ds(baseB, CH)],
                        segf1.at[pl.ds(0, CH)])

        @pl.when(p > 0)
        def _():
            wait_scatter(tok1, osem1)
        pltpu.make_async_copy(tok_hbm.at[idx1], tok1, gsem1).start()

        wait_gather(tok0, gsem0)
        compute_chunk(tok0, segf0, 0)
        pltpu.make_async_copy(tok0, out_hbm.at[pl.ds(baseA, CH)], osem0).start()

        @pl.when(p < B - 1)
        def _():
            baseA2 = pl.multiple_of(baseA + S, CH)
            pltpu.sync_copy(x_hbm.at[pl.ds(baseA2, CH)], idx0)
            pltpu.sync_copy(segf_hbm.at[pl.ds(baseA2, CH)],
                            segf0.at[pl.ds(0, CH)])
            wait_scatter(tok0, osem0)
            pltpu.make_async_copy(tok_hbm.at[idx0], tok0, gsem0).start()

        wait_gather(tok1, gsem1)
        compute_chunk(tok1, segf1, CH)
        pltpu.make_async_copy(tok1, out_hbm.at[pl.ds(baseB, CH)], osem1).start()
        return 0
    lax.fori_loop(0, B, pair_body, 0)

    wait_scatter(tok0, osem0)
    wait_scatter(tok1, osem1)


def kernel(x, seg, tok_embed, pos_embed, seg_embed, gamma, beta):
    x_flat = x.reshape(-1).astype(jnp.int32)
    segf = seg.reshape(-1).astype(jnp.float32)
    out = _emb_kernel(x_flat, segf, tok_embed, pos_embed, seg_embed,
                      gamma, beta)
    return out.reshape(B, S, D)
